# baseline (speedup 1.0000x reference)
import functools

import jax
import jax.numpy as jnp
from jax import lax
from jax.experimental import pallas as pl
from jax.experimental.pallas import tpu as pltpu
from jax.experimental.pallas import tpu_sc as plsc

N = 10000
K = 16
D = 256
LANES = 16
TL = D // LANES
NC = 2
NS = 16
NW = NC * NS
NPW = 320
NP = NW * NPW
C = 2
CK = C * K
CHUNKS = NPW // C
NBUF = 4
GROUPS = CHUNKS // NBUF


def _agg_body(x_hbm, idx_hbm, out_hbm, idx_v, rows_v, agg_v, gsem):
    wid = lax.axis_index("s") * NC + lax.axis_index("c")
    pltpu.sync_copy(idx_hbm.at[wid], idx_v)

    def issue(g, slot, b):
        pltpu.async_copy(
            x_hbm.at[idx_v.at[g, pl.ds(slot * CK, CK)]], rows_v.at[b], gsem)

    def wait(g, slot, b):
        pltpu.make_async_copy(
            x_hbm.at[idx_v.at[g, pl.ds(slot * CK, CK)]], rows_v.at[b], gsem).wait()

    def compute_chunk(c, b):
        def node_body(j, _):
            row0 = j * K
            node = c * C + j
            for t in range(TL):
                col = t * LANES
                s = rows_v[b, row0, pl.ds(col, LANES)]
                for k in range(1, K):
                    s = s + rows_v[b, row0 + k, pl.ds(col, LANES)]
                agg_v[node, pl.ds(col, LANES)] = s
            return 0

        lax.fori_loop(0, C, node_body, 0)

    for b in range(NBUF - 1):
        issue(0, b, b)

    def group_body(i, _):
        c0 = i * NBUF
        for b in range(NBUF):
            c = c0 + b
            wait(i, b, b)
            slot = (b + NBUF - 1) % NBUF
            issue(i if b == 0 else i + 1, slot, slot)
            compute_chunk(c, b)
        return 0

    lax.fori_loop(0, GROUPS - 1, group_body, 0)
    g = GROUPS - 1
    c0 = g * NBUF
    for b in range(NBUF):
        c = c0 + b
        wait(g, b, b)
        if b == 0:
            issue(g, NBUF - 1, NBUF - 1)
        compute_chunk(c, b)
    pltpu.sync_copy(agg_v, out_hbm.at[wid])


@functools.cache
def _agg_call():
    mesh = plsc.VectorSubcoreMesh(core_axis_name="c", subcore_axis_name="s")
    return pl.kernel(
        _agg_body,
        out_type=jax.ShapeDtypeStruct((NW, NPW, D), jnp.float32),
        mesh=mesh,
        scratch_types=[
            pltpu.VMEM((GROUPS, NBUF * CK), jnp.int32),
            pltpu.VMEM((NBUF, CK, D), jnp.float32),
            pltpu.VMEM((NPW, D), jnp.float32),
            pltpu.SemaphoreType.DMA,
        ],
    )


RT = 1000


def _mlp_body(eps_ref, x_ref, agg_ref, w1_ref, b1_ref, w2_ref, b2_ref, o_ref):
    h = (1.0 + eps_ref[0]) * x_ref[...] + agg_ref[...]
    h1 = jnp.dot(h, w1_ref[...], preferred_element_type=jnp.float32) + b1_ref[...]
    h1 = jnp.maximum(h1, 0.0)
    o_ref[...] = jnp.dot(h1, w2_ref[...], preferred_element_type=jnp.float32) + b2_ref[...]


@functools.cache
def _mlp_call():
    return pl.pallas_call(
        _mlp_body,
        grid=(N // RT,),
        in_specs=[
            pl.BlockSpec(memory_space=pltpu.SMEM),
            pl.BlockSpec((RT, D), lambda i: (i, 0)),
            pl.BlockSpec((RT, D), lambda i: (i, 0)),
            pl.BlockSpec((D, D), lambda i: (0, 0)),
            pl.BlockSpec((1, D), lambda i: (0, 0)),
            pl.BlockSpec((D, D), lambda i: (0, 0)),
            pl.BlockSpec((1, D), lambda i: (0, 0)),
        ],
        out_specs=pl.BlockSpec((RT, D), lambda i: (i, 0)),
        out_shape=jax.ShapeDtypeStruct((N, D), jnp.float32),
    )


def kernel(x, neigh, eps, W1, b1, W2, b2):
    x2d = x[0]
    idx = neigh.astype(jnp.int32)
    idx = jnp.pad(idx, ((0, NP - N), (0, 0)))
    idx = idx.reshape(NW, GROUPS, NBUF * CK)
    agg = _agg_call()(x2d, idx).reshape(NP, D)
    eps_arr = jnp.reshape(eps, (1,)).astype(jnp.float32)
    out = _mlp_call()(eps_arr, x2d, agg, W1, jnp.reshape(b1, (1, D)),
                      W2, jnp.reshape(b2, (1, D)))
    return out[None]
